# baseline (speedup 1.0000x reference)
import functools

import jax
import jax.numpy as jnp
from jax import lax
from jax.experimental import pallas as pl
from jax.experimental.pallas import tpu as pltpu
from jax.experimental.pallas import tpu_sc as plsc

MAXLEN = 200
EMBED = 64
SEQ_PER_SUPER = 2


def kernel(x, token_table, pos_table):
    B, T = x.shape
    V, D = token_table.shape
    assert T == MAXLEN and D == EMBED

    info = plsc.get_sparse_core_info()
    nw = info.num_cores * info.num_subcores
    supers_per_w = B // (SEQ_PER_SUPER * nw)

    x32 = x.astype(jnp.int32)

    mesh = plsc.VectorSubcoreMesh(core_axis_name="c", subcore_axis_name="s")

    @functools.partial(
        pl.kernel,
        mesh=mesh,
        out_type=jax.ShapeDtypeStruct((B, T, D), jnp.float32),
        scratch_types=[
            pltpu.VMEM_SHARED((SEQ_PER_SUPER, MAXLEN, D), jnp.float32),
            pltpu.VMEM((SEQ_PER_SUPER, MAXLEN, D), jnp.float32),
            pltpu.VMEM((SEQ_PER_SUPER, MAXLEN, D), jnp.float32),
            pltpu.VMEM((SEQ_PER_SUPER, MAXLEN), jnp.int32),
            pltpu.VMEM((SEQ_PER_SUPER, MAXLEN), jnp.int32),
            pltpu.SemaphoreType.DMA,
            pltpu.SemaphoreType.DMA,
            pltpu.SemaphoreType.DMA,
            pltpu.SemaphoreType.DMA,
        ],
        compiler_params=pltpu.CompilerParams(use_tc_tiling_on_sc=False),
    )
    def sc_kernel(x_hbm, tok_hbm, pos_hbm, out_hbm,
                  posfill, rows0, rows1, idx0, idx1,
                  sem_g0, sem_g1, sem_s0, sem_s1):
        rows = (rows0, rows1)
        idx = (idx0, idx1)
        sem_g = (sem_g0, sem_g1)
        sem_s = (sem_s0, sem_s1)

        cid = lax.axis_index("c")
        sid = lax.axis_index("s")
        wid = sid * info.num_cores + cid
        base = wid * supers_per_w

        @pl.when(sid == 0)
        def _():
            pltpu.sync_copy(pos_hbm, rows0.at[0])
            for rep in range(SEQ_PER_SUPER):
                pltpu.sync_copy(rows0.at[0], posfill.at[rep])
        plsc.subcore_barrier()

        def wait_store(b):
            pltpu.make_async_copy(
                rows[b], out_hbm.at[pl.ds(0, SEQ_PER_SUPER)], sem_s[b]).wait()

        def stage_a(i, b, first_use):
            if not first_use:
                wait_store(b)
            pltpu.sync_copy(posfill, rows[b])
            pltpu.sync_copy(
                x_hbm.at[pl.ds((base + i) * SEQ_PER_SUPER, SEQ_PER_SUPER)],
                idx[b])
            for s in range(SEQ_PER_SUPER):
                pltpu.async_copy(
                    tok_hbm.at[idx[b].at[s]], rows[b].at[s], sem_g[b],
                    add=True)

        def stage_b(i, b):
            for s in range(SEQ_PER_SUPER):
                pltpu.make_async_copy(
                    tok_hbm.at[idx[b].at[s]], rows[b].at[s], sem_g[b]).wait()
            pltpu.async_copy(
                rows[b],
                out_hbm.at[pl.ds((base + i) * SEQ_PER_SUPER, SEQ_PER_SUPER)],
                sem_s[b])

        stage_a(0, 0, True)
        stage_a(1, 1, True)
        stage_b(0, 0)

        @pl.loop(0, (supers_per_w - 2) // 2)
        def _(t):
            i = 2 + 2 * t
            stage_a(i, 0, False)
            stage_b(i - 1, 1)
            stage_a(i + 1, 1, False)
            stage_b(i, 0)

        stage_b(supers_per_w - 1, 1)
        wait_store(0)
        wait_store(1)

    return sc_kernel(x32, token_table, pos_table)
